# Initial kernel scaffold; baseline (speedup 1.0000x reference)
import jax
import jax.numpy as jnp
from jax.experimental import pallas as pl


def kernel(x, edge_index, W, b):
    raise NotImplementedError("write your pallas kernel here")


# speedup vs baseline: 1.5752x; 1.5752x over previous
import functools

import jax
import jax.numpy as jnp
from jax import lax
from jax.experimental import pallas as pl
from jax.experimental.pallas import tpu as pltpu
from jax.experimental.pallas import tpu_sc as plsc

N = 10000
C = 128
OUT = 128
K = 32

NW = 32
NPAD = 10240
PER_W = NPAD // NW
CHUNK = 8
NCHUNK = PER_W // CHUNK
LG = OUT // 16


def _mm_body(xs_ref, w_ref, b_ref, a_ref, t_ref):
    xb = xs_ref[...]
    w = w_ref[...]
    w1 = w[:, :C]
    w2 = w[:, C:]
    dn = (((1,), (1,)), ((), ()))
    a_ref[...] = (
        lax.dot_general(xb, w1 - w2, dn, preferred_element_type=jnp.float32)
        + b_ref[...]
    )
    t_ref[...] = lax.dot_general(xb, w2, dn, preferred_element_type=jnp.float32)


_MM_BLK = 1024


def _mm(xs, w, b2):
    return pl.pallas_call(
        _mm_body,
        grid=(NPAD // _MM_BLK,),
        in_specs=[
            pl.BlockSpec((_MM_BLK, C), lambda i: (i, 0)),
            pl.BlockSpec((OUT, 2 * C), lambda i: (0, 0)),
            pl.BlockSpec((1, OUT), lambda i: (0, 0)),
        ],
        out_specs=[
            pl.BlockSpec((_MM_BLK, OUT), lambda i: (i, 0)),
            pl.BlockSpec((_MM_BLK, OUT), lambda i: (i, 0)),
        ],
        out_shape=[
            jax.ShapeDtypeStruct((NPAD, OUT), jnp.float32),
            jax.ShapeDtypeStruct((NPAD, OUT), jnp.float32),
        ],
    )(xs, w, b2)


@functools.partial(
    pl.kernel,
    out_type=jax.ShapeDtypeStruct((NPAD, OUT), jnp.float32),
    mesh=plsc.VectorSubcoreMesh(core_axis_name="c", subcore_axis_name="s"),
    scratch_types=[
        pltpu.VMEM((PER_W * K,), jnp.int32),
        pltpu.VMEM((CHUNK * K, OUT), jnp.float32),
        pltpu.VMEM((CHUNK, OUT), jnp.float32),
        pltpu.VMEM((CHUNK, OUT), jnp.float32),
        pltpu.SemaphoreType.DMA,
    ],
)
def _sc_gather_max(t_hbm, idx_hbm, a_hbm, out_hbm, idx_v, gbuf, abuf, obuf, sem):
    wid = lax.axis_index("s") * 2 + lax.axis_index("c")
    base = wid * PER_W
    pltpu.sync_copy(idx_hbm.at[pl.ds(base * K, PER_W * K)], idx_v)

    def chunk_body(ch, carry):
        gb = base + ch * CHUNK
        cp = pltpu.async_copy(
            t_hbm.at[idx_v.at[pl.ds(ch * (CHUNK * K), CHUNK * K)]], gbuf, sem
        )
        pltpu.sync_copy(a_hbm.at[pl.ds(gb, CHUNK)], abuf)
        cp.wait()

        def node_body(n, carry2):
            rb = n * K
            for g in range(LG):
                sl = pl.ds(g * 16, 16)
                acc = gbuf[rb, sl]
                for k in range(1, K):
                    acc = jnp.maximum(acc, gbuf[rb + k, sl])
                obuf[n, sl] = jnp.maximum(acc + abuf[n, sl], 0.0)
            return carry2

        lax.fori_loop(0, CHUNK, node_body, 0)
        pltpu.sync_copy(obuf, out_hbm.at[pl.ds(gb, CHUNK)])
        return carry

    lax.fori_loop(0, NCHUNK, chunk_body, 0)


def kernel(x, edge_index, W, b):
    x = x.astype(jnp.float32)
    idx = edge_index[0].astype(jnp.int32).reshape(N * K)
    idx = jnp.pad(idx, (0, (NPAD - N) * K))
    xs = jnp.concatenate(
        [x, jnp.full((1, C), 1e6, jnp.float32)], axis=0
    )
    xs = jnp.pad(xs, ((0, NPAD - N - 1), (0, 0)))
    a, t = _mm(xs, W.astype(jnp.float32), b.astype(jnp.float32).reshape(1, OUT))
    out = _sc_gather_max(t, idx, a)
    return out[:N]


# speedup vs baseline: 1.7528x; 1.1127x over previous
import functools

import jax
import jax.numpy as jnp
from jax import lax
from jax.experimental import pallas as pl
from jax.experimental.pallas import tpu as pltpu
from jax.experimental.pallas import tpu_sc as plsc

N = 10000
C = 128
OUT = 128
K = 32

NW = 32
NPAD = 10240
PER_W = NPAD // NW
CHUNK = 8
NCHUNK = PER_W // CHUNK
LG = OUT // 16


def _mm_body(xs_ref, w_ref, b_ref, a_ref, t_ref):
    xb = xs_ref[...]
    w = w_ref[...]
    w1 = w[:, :C]
    w2 = w[:, C:]
    dn = (((1,), (1,)), ((), ()))
    a_ref[...] = (
        lax.dot_general(xb, w1 - w2, dn, preferred_element_type=jnp.float32)
        + b_ref[...]
    )
    t_ref[...] = lax.dot_general(xb, w2, dn, preferred_element_type=jnp.float32)


_MM_BLK = 1024


def _mm(xs, w, b2):
    return pl.pallas_call(
        _mm_body,
        grid=(NPAD // _MM_BLK,),
        in_specs=[
            pl.BlockSpec((_MM_BLK, C), lambda i: (i, 0)),
            pl.BlockSpec((OUT, 2 * C), lambda i: (0, 0)),
            pl.BlockSpec((1, OUT), lambda i: (0, 0)),
        ],
        out_specs=[
            pl.BlockSpec((_MM_BLK, OUT), lambda i: (i, 0)),
            pl.BlockSpec((_MM_BLK, OUT), lambda i: (i, 0)),
        ],
        out_shape=[
            jax.ShapeDtypeStruct((NPAD, OUT), jnp.float32),
            jax.ShapeDtypeStruct((NPAD, OUT), jnp.float32),
        ],
    )(xs, w, b2)


@functools.partial(
    pl.kernel,
    out_type=jax.ShapeDtypeStruct((NPAD, OUT), jnp.float32),
    mesh=plsc.VectorSubcoreMesh(core_axis_name="c", subcore_axis_name="s"),
    scratch_types=[
        pltpu.VMEM((PER_W * K,), jnp.int32),
        pltpu.VMEM((CHUNK * K, OUT), jnp.float32),
        pltpu.VMEM((CHUNK * K, OUT), jnp.float32),
        pltpu.VMEM((CHUNK, OUT), jnp.float32),
        pltpu.VMEM((CHUNK, OUT), jnp.float32),
        pltpu.VMEM((PER_W, OUT), jnp.float32),
        pltpu.SemaphoreType.DMA,
        pltpu.SemaphoreType.DMA,
        pltpu.SemaphoreType.DMA,
        pltpu.SemaphoreType.DMA,
    ],
)
def _sc_gather_max(
    t_hbm, idx_hbm, a_hbm, out_hbm,
    idx_v, gbuf0, gbuf1, abuf0, abuf1, oblk,
    gsem0, gsem1, asem0, asem1,
):
    wid = lax.axis_index("s") * 2 + lax.axis_index("c")
    base = wid * PER_W
    pltpu.sync_copy(idx_hbm.at[pl.ds(base * K, PER_W * K)], idx_v)

    gbufs = (gbuf0, gbuf1)
    abufs = (abuf0, abuf1)
    gsems = (gsem0, gsem1)
    asems = (asem0, asem1)

    def start(ch, slot):
        pltpu.async_copy(
            t_hbm.at[idx_v.at[pl.ds(ch * (CHUNK * K), CHUNK * K)]],
            gbufs[slot], gsems[slot],
        )
        pltpu.async_copy(
            a_hbm.at[pl.ds(base + ch * CHUNK, CHUNK)], abufs[slot], asems[slot]
        )

    def finish(ch, slot):
        gbuf, abuf = gbufs[slot], abufs[slot]
        pltpu.make_async_copy(
            t_hbm.at[idx_v.at[pl.ds(ch * (CHUNK * K), CHUNK * K)]],
            gbuf, gsems[slot],
        ).wait()
        pltpu.make_async_copy(
            a_hbm.at[pl.ds(base + ch * CHUNK, CHUNK)], abuf, asems[slot]
        ).wait()

        def node_body(n, carry2):
            rb = n * K
            orow = ch * CHUNK + n
            for g in range(LG):
                sl = pl.ds(g * 16, 16)
                acc = gbuf[rb, sl]
                for k in range(1, K):
                    acc = jnp.maximum(acc, gbuf[rb + k, sl])
                oblk[orow, sl] = jnp.maximum(acc + abuf[n, sl], 0.0)
            return carry2

        lax.fori_loop(0, CHUNK, node_body, 0)

    start(0, 0)

    def pair_body(p, carry):
        ch0 = 2 * p
        start(ch0 + 1, 1)
        finish(ch0, 0)
        start(ch0 + 2, 0)
        finish(ch0 + 1, 1)
        return carry

    lax.fori_loop(0, NCHUNK // 2 - 1, pair_body, 0)
    start(NCHUNK - 1, 1)
    finish(NCHUNK - 2, 0)
    finish(NCHUNK - 1, 1)

    pltpu.sync_copy(oblk, out_hbm.at[pl.ds(base, PER_W)])


def kernel(x, edge_index, W, b):
    x = x.astype(jnp.float32)
    idx = edge_index[0].astype(jnp.int32).reshape(N * K)
    idx = jnp.pad(idx, (0, (NPAD - N) * K))
    xs = jnp.concatenate(
        [x, jnp.full((1, C), 1e6, jnp.float32)], axis=0
    )
    xs = jnp.pad(xs, ((0, NPAD - N - 1), (0, 0)))
    a, t = _mm(xs, W.astype(jnp.float32), b.astype(jnp.float32).reshape(1, OUT))
    out = _sc_gather_max(t, idx, a)
    return out[:N]


# speedup vs baseline: 5.8364x; 3.3298x over previous
import functools

import jax
import jax.numpy as jnp
from jax import lax
from jax.experimental import pallas as pl
from jax.experimental.pallas import tpu as pltpu
from jax.experimental.pallas import tpu_sc as plsc

N = 10000
C = 128
OUT = 128
K = 32

NW = 32
NPAD = 10240
PER_W = NPAD // NW
CHUNK = 4
NCHUNK = PER_W // CHUNK
LG = OUT // 16


def _mm_body(xs_ref, w_ref, b_ref, a_ref, t_ref):
    xb = xs_ref[...]
    w = w_ref[...]
    w1 = w[:, :C]
    w2 = w[:, C:]
    dn = (((1,), (1,)), ((), ()))
    a_ref[...] = (
        lax.dot_general(xb, w1 - w2, dn, preferred_element_type=jnp.float32)
        + b_ref[...]
    )
    t_ref[...] = lax.dot_general(xb, w2, dn, preferred_element_type=jnp.float32)


_MM_BLK = 1024


def _mm(xs, w, b2):
    return pl.pallas_call(
        _mm_body,
        grid=(NPAD // _MM_BLK,),
        in_specs=[
            pl.BlockSpec((_MM_BLK, C), lambda i: (i, 0)),
            pl.BlockSpec((OUT, 2 * C), lambda i: (0, 0)),
            pl.BlockSpec((1, OUT), lambda i: (0, 0)),
        ],
        out_specs=[
            pl.BlockSpec((_MM_BLK, OUT), lambda i: (i, 0)),
            pl.BlockSpec((_MM_BLK, OUT), lambda i: (i, 0)),
        ],
        out_shape=[
            jax.ShapeDtypeStruct((NPAD, OUT), jnp.float32),
            jax.ShapeDtypeStruct((NPAD, OUT), jnp.float32),
        ],
    )(xs, w, b2)


@functools.partial(
    pl.kernel,
    out_type=jax.ShapeDtypeStruct((NPAD, OUT), jnp.float32),
    mesh=plsc.VectorSubcoreMesh(core_axis_name="c", subcore_axis_name="s"),
    scratch_types=[
        pltpu.VMEM_SHARED((NPAD, OUT), jnp.float32),
        pltpu.VMEM((PER_W * K,), jnp.int32),
        pltpu.VMEM((CHUNK * K, OUT), jnp.float32),
        pltpu.VMEM((CHUNK * K, OUT), jnp.float32),
        pltpu.VMEM((CHUNK, OUT), jnp.float32),
        pltpu.VMEM((CHUNK, OUT), jnp.float32),
        pltpu.VMEM((CHUNK, OUT), jnp.float32),
        pltpu.VMEM((CHUNK, OUT), jnp.float32),
        pltpu.SemaphoreType.DMA,
        pltpu.SemaphoreType.DMA,
        pltpu.SemaphoreType.DMA,
        pltpu.SemaphoreType.DMA,
        pltpu.SemaphoreType.DMA,
        pltpu.SemaphoreType.DMA,
    ],
)
def _sc_gather_max(
    t_hbm, idx_hbm, a_hbm, out_hbm,
    t_sh, idx_v, gbuf0, gbuf1, abuf0, abuf1, obuf0, obuf1,
    gsem0, gsem1, asem0, asem1, osem0, osem1,
):
    sid = lax.axis_index("s")
    wid = sid * 2 + lax.axis_index("c")
    base = wid * PER_W
    rows = NPAD // 16
    pltpu.sync_copy(
        t_hbm.at[pl.ds(sid * rows, rows)], t_sh.at[pl.ds(sid * rows, rows)]
    )
    pltpu.sync_copy(idx_hbm.at[pl.ds(base * K, PER_W * K)], idx_v)
    plsc.subcore_barrier()

    gbufs = (gbuf0, gbuf1)
    abufs = (abuf0, abuf1)
    obufs = (obuf0, obuf1)
    gsems = (gsem0, gsem1)
    asems = (asem0, asem1)
    osems = (osem0, osem1)

    def start(ch, slot):
        pltpu.async_copy(
            t_sh.at[idx_v.at[pl.ds(ch * (CHUNK * K), CHUNK * K)]],
            gbufs[slot], gsems[slot],
        )
        pltpu.async_copy(
            a_hbm.at[pl.ds(base + ch * CHUNK, CHUNK)], abufs[slot], asems[slot]
        )

    def finish(ch, slot, wait_store):
        gbuf, abuf, obuf = gbufs[slot], abufs[slot], obufs[slot]
        pltpu.make_async_copy(
            t_sh.at[idx_v.at[pl.ds(ch * (CHUNK * K), CHUNK * K)]],
            gbuf, gsems[slot],
        ).wait()
        pltpu.make_async_copy(
            a_hbm.at[pl.ds(base + ch * CHUNK, CHUNK)], abuf, asems[slot]
        ).wait()
        if wait_store:
            pltpu.make_async_copy(
                obuf, out_hbm.at[pl.ds(base, CHUNK)], osems[slot]
            ).wait()

        def node_body(n, carry2):
            rb = n * K
            for g in range(LG):
                sl = pl.ds(g * 16, 16)
                acc = gbuf[rb, sl]
                for k in range(1, K):
                    acc = jnp.maximum(acc, gbuf[rb + k, sl])
                obuf[n, sl] = jnp.maximum(acc + abuf[n, sl], 0.0)
            return carry2

        lax.fori_loop(0, CHUNK, node_body, 0)
        pltpu.async_copy(
            obuf, out_hbm.at[pl.ds(base + ch * CHUNK, CHUNK)], osems[slot]
        )

    start(0, 0)
    start(1, 1)
    finish(0, 0, False)
    start(2, 0)
    finish(1, 1, False)
    start(3, 1)

    def pair_body(p, carry):
        ch0 = 2 * p
        finish(ch0, 0, True)
        start(ch0 + 2, 0)
        finish(ch0 + 1, 1, True)
        start(ch0 + 3, 1)
        return carry

    lax.fori_loop(1, NCHUNK // 2 - 1, pair_body, 0)
    finish(NCHUNK - 2, 0, True)
    finish(NCHUNK - 1, 1, True)
    pltpu.make_async_copy(obuf0, out_hbm.at[pl.ds(base, CHUNK)], osem0).wait()
    pltpu.make_async_copy(obuf1, out_hbm.at[pl.ds(base, CHUNK)], osem1).wait()


def kernel(x, edge_index, W, b):
    x = x.astype(jnp.float32)
    idx = edge_index[0].astype(jnp.int32).reshape(N * K)
    idx = jnp.pad(idx, (0, (NPAD - N) * K))
    xs = jnp.concatenate(
        [x, jnp.full((1, C), 1e6, jnp.float32)], axis=0
    )
    xs = jnp.pad(xs, ((0, NPAD - N - 1), (0, 0)))
    a, t = _mm(xs, W.astype(jnp.float32), b.astype(jnp.float32).reshape(1, OUT))
    out = _sc_gather_max(t, idx, a)
    return out[:N]
